# baseline (speedup 1.0000x reference)
import functools

import jax
import jax.numpy as jnp
from jax import lax
from jax.experimental import pallas as pl
from jax.experimental.pallas import tpu as pltpu
from jax.experimental.pallas import tpu_sc as plsc

NC, NS = 2, 16
NW = NC * NS


def kernel(f, idx, fb, idx_bank):
    f2 = f.reshape(-1, f.shape[-1])
    idx2 = idx.reshape(-1)
    N, F = f2.shape
    S = fb.shape[0]
    rest = S - N
    fper = N // NW
    per = rest // NW // 8 * 8
    per_i = rest // NW // 128 * 128
    left = rest - NW * per
    left_i = rest - NW * per_i
    mesh = plsc.VectorSubcoreMesh(
        core_axis_name="c", subcore_axis_name="s",
        num_cores=NC, num_subcores=NS,
    )

    @functools.partial(
        pl.kernel,
        out_type=[
            jax.ShapeDtypeStruct((S, F), fb.dtype),
            jax.ShapeDtypeStruct((S,), idx_bank.dtype),
        ],
        mesh=mesh,
        scratch_types=[
            pltpu.SemaphoreType.DMA,
            pltpu.VMEM((left_i,), idx_bank.dtype),
        ],
        compiler_params=pltpu.CompilerParams(use_tc_tiling_on_sc=True),
    )
    def k(f_h, idx_h, fb_h, idxb_h, out_h, idxo_h, sem, ibuf):
        c = lax.axis_index("c")
        s = lax.axis_index("s")
        w = s * NC + c
        copies = [
            pltpu.make_async_copy(
                f_h.at[pl.ds(w * fper, fper)],
                out_h.at[pl.ds(w * fper, fper)], sem),
            pltpu.make_async_copy(
                fb_h.at[pl.ds(w * per, per)],
                out_h.at[pl.ds(N + w * per, per)], sem),
            pltpu.make_async_copy(
                idx_h.at[pl.ds(w * fper, fper)],
                idxo_h.at[pl.ds(w * fper, fper)], sem),
            pltpu.make_async_copy(
                idxb_h.at[pl.ds(w * per_i, per_i)],
                idxo_h.at[pl.ds(N + w * per_i, per_i)], sem),
        ]
        for cp in copies:
            cp.start()

        @pl.when(w == 0)
        def _():
            extra = [
                pltpu.make_async_copy(
                    fb_h.at[pl.ds(NW * per, left)],
                    out_h.at[pl.ds(N + NW * per, left)], sem),
            ]
            for cp in extra:
                cp.start()
            pltpu.sync_copy(idxb_h.at[pl.ds(NW * per_i, left_i)], ibuf)
            pltpu.sync_copy(ibuf, idxo_h.at[pl.ds(N + NW * per_i, left_i)])
            for cp in extra:
                cp.wait()

        for cp in copies:
            cp.wait()

    out_fb, out_idx = k(f2, idx2, fb, idx_bank)
    return (out_fb, out_idx)


# speedup vs baseline: 15.1743x; 15.1743x over previous
import functools

import jax
import jax.numpy as jnp
from jax import lax
from jax.experimental import pallas as pl
from jax.experimental.pallas import tpu as pltpu
from jax.experimental.pallas import tpu_sc as plsc

NC, NS = 2, 16
NW = NC * NS

CHR = 400
F_SUB = 128
IDX_CH = 3840


def kernel(f, idx, fb, idx_bank):
    f2 = f.reshape(-1, f.shape[-1])
    idx2 = idx.reshape(-1)
    N, F = f2.shape
    S = fb.shape[0]
    rest = S - N

    fper = N // NW
    per = rest // NW // 8 * 8
    left = rest - NW * per
    nch = per // CHR
    tail = per - nch * CHR
    pairs = nch // 2
    assert nch % 2 == 0 and fper % F_SUB == 0 and tail % 8 == 0

    per_i = rest // NW // 128 * 128
    nich = per_i // IDX_CH
    itail = per_i - nich * IDX_CH
    left_i = rest - NW * per_i

    mesh = plsc.VectorSubcoreMesh(
        core_axis_name="c", subcore_axis_name="s",
        num_cores=NC, num_subcores=NS,
    )

    @functools.partial(
        pl.kernel,
        out_type=[
            jax.ShapeDtypeStruct((S, F), fb.dtype),
            jax.ShapeDtypeStruct((S,), idx_bank.dtype),
        ],
        mesh=mesh,
        compiler_params=pltpu.CompilerParams(use_tc_tiling_on_sc=True),
        scratch_types=[
            pltpu.VMEM((CHR, F), fb.dtype),
            pltpu.VMEM((CHR, F), fb.dtype),
            pltpu.VMEM((F_SUB, F), fb.dtype),
            pltpu.VMEM((IDX_CH,), idx_bank.dtype),
            pltpu.VMEM((IDX_CH,), idx_bank.dtype),
            pltpu.SemaphoreType.DMA((2,)),
        ],
    )
    def k(f_h, idx_h, fb_h, idxb_h, out_h, idxo_h,
          buf0, buf1, fbuf, ibuf0, ibuf1, gsem):
        bufs = (buf0, buf1)
        ibufs = (ibuf0, ibuf1)
        c_ax = lax.axis_index("c")
        s_ax = lax.axis_index("s")
        w = s_ax * NC + c_ax
        fb_base = w * per
        ob_base = N + w * per

        for u in range(fper // F_SUB):
            r0 = w * fper + u * F_SUB
            pltpu.sync_copy(f_h.at[pl.ds(r0, F_SUB)], fbuf)
            pltpu.sync_copy(fbuf, out_h.at[pl.ds(r0, F_SUB)])

        def g_copy(c, b):
            return pltpu.make_async_copy(
                fb_h.at[pl.ds(fb_base + c * CHR, CHR)], bufs[b], gsem.at[b])

        g_copy(0, 0).start()
        g_copy(1, 1).start()

        def pair_body(p, carry):
            for b in range(2):
                c = 2 * p + b
                g_copy(c, b).wait()
                pltpu.sync_copy(
                    bufs[b], out_h.at[pl.ds(ob_base + c * CHR, CHR)])

                @pl.when(c + 2 < nch)
                def _():
                    g_copy(c + 2, b).start()

                @pl.when(c + 2 == nch)
                def _():
                    pltpu.make_async_copy(
                        fb_h.at[pl.ds(fb_base + nch * CHR, tail)],
                        bufs[b].at[pl.ds(0, tail)], gsem.at[b]).start()
            return carry

        lax.fori_loop(0, pairs, pair_body, 0)

        b_t = nch % 2
        pltpu.make_async_copy(
            fb_h.at[pl.ds(fb_base + nch * CHR, tail)],
            bufs[b_t].at[pl.ds(0, tail)], gsem.at[b_t]).wait()
        pltpu.sync_copy(
            bufs[b_t].at[pl.ds(0, tail)],
            out_h.at[pl.ds(ob_base + nch * CHR, tail)])

        pltpu.sync_copy(idx_h.at[pl.ds(w * fper, fper)],
                        ibuf0.at[pl.ds(0, fper)])
        pltpu.sync_copy(ibuf0.at[pl.ds(0, fper)],
                        idxo_h.at[pl.ds(w * fper, fper)])

        ib_base = w * per_i
        for ci in range(nich):
            b = ci % 2
            pltpu.sync_copy(
                idxb_h.at[pl.ds(ib_base + ci * IDX_CH, IDX_CH)], ibufs[b])
            pltpu.sync_copy(
                ibufs[b], idxo_h.at[pl.ds(N + ib_base + ci * IDX_CH, IDX_CH)])
        if itail:
            b = nich % 2
            pltpu.sync_copy(
                idxb_h.at[pl.ds(ib_base + nich * IDX_CH, itail)],
                ibufs[b].at[pl.ds(0, itail)])
            pltpu.sync_copy(
                ibufs[b].at[pl.ds(0, itail)],
                idxo_h.at[pl.ds(N + ib_base + nich * IDX_CH, itail)])

        @pl.when(w == 0)
        def _():
            pltpu.sync_copy(fb_h.at[pl.ds(NW * per, left)],
                            fbuf.at[pl.ds(0, left)])
            pltpu.sync_copy(fbuf.at[pl.ds(0, left)],
                            out_h.at[pl.ds(N + NW * per, left)])
            pltpu.sync_copy(idxb_h.at[pl.ds(NW * per_i, left_i)],
                            ibuf0.at[pl.ds(0, left_i)])
            pltpu.sync_copy(ibuf0.at[pl.ds(0, left_i)],
                            idxo_h.at[pl.ds(N + NW * per_i, left_i)])

    out_fb, out_idx = k(f2, idx2, fb, idx_bank)
    return (out_fb, out_idx)


# speedup vs baseline: 15.1868x; 1.0008x over previous
import functools

import jax
import jax.numpy as jnp
from jax import lax
from jax.experimental import pallas as pl
from jax.experimental.pallas import tpu as pltpu
from jax.experimental.pallas import tpu_sc as plsc

NC, NS = 2, 16
NW = NC * NS

NBUF = 4
LOOK = 2
CHR = 240
IDX_CH = 3840


def kernel(f, idx, fb, idx_bank):
    f2 = f.reshape(-1, f.shape[-1])
    idx2 = idx.reshape(-1)
    N, F = f2.shape
    S = fb.shape[0]
    rest = S - N

    fper = N // NW
    per = rest // NW // 8 * 8
    left = rest - NW * per
    nch = per // CHR // NBUF * NBUF
    groups = nch // NBUF
    tail = per - nch * CHR
    assert 0 <= tail <= CHR and tail % 8 == 0

    per_i = rest // NW // 128 * 128
    nich = per_i // IDX_CH
    assert nich * IDX_CH == per_i
    left_i = rest - NW * per_i

    f_chunks = []
    off = 0
    while off < fper:
        sz = min(CHR, fper - off)
        f_chunks.append((off, sz))
        off += sz

    mesh = plsc.VectorSubcoreMesh(
        core_axis_name="c", subcore_axis_name="s",
        num_cores=NC, num_subcores=NS,
    )

    @functools.partial(
        pl.kernel,
        out_type=[
            jax.ShapeDtypeStruct((S, F), fb.dtype),
            jax.ShapeDtypeStruct((S,), idx_bank.dtype),
        ],
        mesh=mesh,
        compiler_params=pltpu.CompilerParams(use_tc_tiling_on_sc=True),
        scratch_types=(
            [pltpu.VMEM((CHR, F), fb.dtype)] * NBUF
            + [pltpu.VMEM((IDX_CH,), idx_bank.dtype)] * 2
            + [pltpu.SemaphoreType.DMA((NBUF,)),
               pltpu.SemaphoreType.DMA((NBUF,))]
        ),
    )
    def k(f_h, idx_h, fb_h, idxb_h, out_h, idxo_h,
          buf0, buf1, buf2, buf3, ibuf0, ibuf1, gsem, ssem):
        bufs = (buf0, buf1, buf2, buf3)
        ibufs = (ibuf0, ibuf1)
        c_ax = lax.axis_index("c")
        s_ax = lax.axis_index("s")
        w = s_ax * NC + c_ax
        fb_base = w * per
        ob_base = N + w * per

        for off, sz in f_chunks:
            r0 = w * fper + off
            pltpu.sync_copy(f_h.at[pl.ds(r0, sz)], buf0.at[pl.ds(0, sz)])
            pltpu.sync_copy(buf0.at[pl.ds(0, sz)], out_h.at[pl.ds(r0, sz)])

        def g_copy(c, b):
            return pltpu.make_async_copy(
                fb_h.at[pl.ds(fb_base + c * CHR, CHR)], bufs[b], gsem.at[b])

        def s_copy(c, b):
            return pltpu.make_async_copy(
                bufs[b], out_h.at[pl.ds(ob_base + c * CHR, CHR)], ssem.at[b])

        def g_tail(b):
            return pltpu.make_async_copy(
                fb_h.at[pl.ds(fb_base + nch * CHR, tail)],
                bufs[b].at[pl.ds(0, tail)], gsem.at[b])

        g_copy(0, 0).start()
        g_copy(1, 1).start()

        def group_body(g, carry):
            for b in range(NBUF):
                c = NBUF * g + b
                g_copy(c, b).wait()
                s_copy(c, b).start()
                j = c + LOOK
                bj = (b + LOOK) % NBUF

                @pl.when(j < nch)
                def _():
                    @pl.when(j >= NBUF)
                    def _():
                        s_copy(j - NBUF, bj).wait()
                    g_copy(j, bj).start()

                if tail:
                    @pl.when(j == nch)
                    def _():
                        s_copy(j - NBUF, bj).wait()
                        g_tail(bj).start()
            return carry

        lax.fori_loop(0, groups, group_body, 0)

        b_t = nch % NBUF
        if tail:
            g_tail(b_t).wait()
            pltpu.make_async_copy(
                bufs[b_t].at[pl.ds(0, tail)],
                out_h.at[pl.ds(ob_base + nch * CHR, tail)],
                ssem.at[b_t]).start()
        drain_from = nch - 3 if tail else nch - 4
        for c in range(drain_from, nch):
            s_copy(c, c % NBUF).wait()
        if tail:
            pltpu.make_async_copy(
                bufs[b_t].at[pl.ds(0, tail)],
                out_h.at[pl.ds(ob_base + nch * CHR, tail)],
                ssem.at[b_t]).wait()

        pltpu.sync_copy(idx_h.at[pl.ds(w * fper, fper)],
                        ibuf0.at[pl.ds(0, fper)])
        pltpu.sync_copy(ibuf0.at[pl.ds(0, fper)],
                        idxo_h.at[pl.ds(w * fper, fper)])

        ib_base = w * per_i
        for ci in range(nich):
            b = ci % 2
            pltpu.sync_copy(
                idxb_h.at[pl.ds(ib_base + ci * IDX_CH, IDX_CH)], ibufs[b])
            pltpu.sync_copy(
                ibufs[b], idxo_h.at[pl.ds(N + ib_base + ci * IDX_CH, IDX_CH)])

        @pl.when(w == 0)
        def _():
            pltpu.sync_copy(fb_h.at[pl.ds(NW * per, left)],
                            buf0.at[pl.ds(0, left)])
            pltpu.sync_copy(buf0.at[pl.ds(0, left)],
                            out_h.at[pl.ds(N + NW * per, left)])
            pltpu.sync_copy(idxb_h.at[pl.ds(NW * per_i, left_i)],
                            ibuf0.at[pl.ds(0, left_i)])
            pltpu.sync_copy(ibuf0.at[pl.ds(0, left_i)],
                            idxo_h.at[pl.ds(N + NW * per_i, left_i)])

    out_fb, out_idx = k(f2, idx2, fb, idx_bank)
    return (out_fb, out_idx)


# speedup vs baseline: 15.3482x; 1.0106x over previous
import functools

import jax
import jax.numpy as jnp
from jax import lax
from jax.experimental import pallas as pl
from jax.experimental.pallas import tpu as pltpu
from jax.experimental.pallas import tpu_sc as plsc

NC, NS = 2, 16
NW = NC * NS

NBUF = 4
LOOK = 2
CHR = 240
IDX_BLK = 16384


def _fb_sc_call(f2, idx2, fb, idx_bank):
    N, F = f2.shape
    S = fb.shape[0]
    rest = S - N

    fper = N // NW
    per = rest // NW // 8 * 8
    left = rest - NW * per
    nch = per // CHR // NBUF * NBUF
    groups = nch // NBUF
    tail = per - nch * CHR
    assert 0 < tail <= CHR and tail % 8 == 0

    f_chunks = []
    off = 0
    while off < fper:
        sz = min(CHR, fper - off)
        f_chunks.append((off, sz))
        off += sz

    mesh = plsc.VectorSubcoreMesh(
        core_axis_name="c", subcore_axis_name="s",
        num_cores=NC, num_subcores=NS,
    )

    @functools.partial(
        pl.kernel,
        out_type=jax.ShapeDtypeStruct((S, F), fb.dtype),
        mesh=mesh,
        scratch_types=(
            [pltpu.VMEM((CHR, F), fb.dtype)] * NBUF
            + [pltpu.SemaphoreType.DMA((NBUF,)),
               pltpu.SemaphoreType.DMA((NBUF,))]
        ),
    )
    def k(f_h, fb_h, out_h, buf0, buf1, buf2, buf3, gsem, ssem):
        bufs = (buf0, buf1, buf2, buf3)
        c_ax = lax.axis_index("c")
        s_ax = lax.axis_index("s")
        w = s_ax * NC + c_ax
        fb_base = w * per
        ob_base = N + w * per

        for off, sz in f_chunks:
            r0 = w * fper + off
            pltpu.sync_copy(f_h.at[pl.ds(r0, sz)], buf0.at[pl.ds(0, sz)])
            pltpu.sync_copy(buf0.at[pl.ds(0, sz)], out_h.at[pl.ds(r0, sz)])

        def g_copy(c, b):
            return pltpu.make_async_copy(
                fb_h.at[pl.ds(fb_base + c * CHR, CHR)], bufs[b], gsem.at[b])

        def s_copy(c, b):
            return pltpu.make_async_copy(
                bufs[b], out_h.at[pl.ds(ob_base + c * CHR, CHR)], ssem.at[b])

        def g_tail(b):
            return pltpu.make_async_copy(
                fb_h.at[pl.ds(fb_base + nch * CHR, tail)],
                bufs[b].at[pl.ds(0, tail)], gsem.at[b])

        g_copy(0, 0).start()
        g_copy(1, 1).start()

        def group_body(g, carry):
            for b in range(NBUF):
                c = NBUF * g + b
                g_copy(c, b).wait()
                s_copy(c, b).start()
                j = c + LOOK
                bj = (b + LOOK) % NBUF

                @pl.when(j < nch)
                def _():
                    @pl.when(j >= NBUF)
                    def _():
                        s_copy(j - NBUF, bj).wait()
                    g_copy(j, bj).start()

                @pl.when(j == nch)
                def _():
                    s_copy(j - NBUF, bj).wait()
                    g_tail(bj).start()
            return carry

        lax.fori_loop(0, groups, group_body, 0)

        b_t = nch % NBUF
        g_tail(b_t).wait()
        pltpu.make_async_copy(
            bufs[b_t].at[pl.ds(0, tail)],
            out_h.at[pl.ds(ob_base + nch * CHR, tail)],
            ssem.at[b_t]).start()
        for c in range(nch - 3, nch):
            s_copy(c, c % NBUF).wait()
        pltpu.make_async_copy(
            bufs[b_t].at[pl.ds(0, tail)],
            out_h.at[pl.ds(ob_base + nch * CHR, tail)],
            ssem.at[b_t]).wait()

        @pl.when(w == 0)
        def _():
            pltpu.sync_copy(fb_h.at[pl.ds(NW * per, left)],
                            buf0.at[pl.ds(0, left)])
            pltpu.sync_copy(buf0.at[pl.ds(0, left)],
                            out_h.at[pl.ds(N + NW * per, left)])

    return k(f2, fb)


def _idx_body(idx_ref, idxb_ref, idxo_ref):
    i = pl.program_id(0)

    @pl.when(i == 0)
    def _():
        idxo_ref[...] = idx_ref[...]

    @pl.when(i > 0)
    def _():
        idxo_ref[...] = idxb_ref[...]


def _idx_tc_call(idx2, idx_bank):
    (N,) = idx2.shape
    (S,) = idx_bank.shape
    assert N == IDX_BLK
    nidx = pl.cdiv(S, IDX_BLK)
    return pl.pallas_call(
        _idx_body,
        grid=(nidx,),
        in_specs=[
            pl.BlockSpec((IDX_BLK,), lambda i: (0,)),
            pl.BlockSpec((IDX_BLK,), lambda i: (jnp.maximum(i - 1, 0),)),
        ],
        out_specs=pl.BlockSpec((IDX_BLK,), lambda i: (i,)),
        out_shape=jax.ShapeDtypeStruct((S,), idx_bank.dtype),
    )(idx2, idx_bank)


def kernel(f, idx, fb, idx_bank):
    f2 = f.reshape(-1, f.shape[-1])
    idx2 = idx.reshape(-1)
    out_fb = _fb_sc_call(f2, idx2, fb, idx_bank)
    out_idx = _idx_tc_call(idx2, idx_bank)
    return (out_fb, out_idx)


# speedup vs baseline: 15.9219x; 1.0374x over previous
import functools

import jax
import jax.numpy as jnp
from jax.experimental import pallas as pl
from jax.experimental.pallas import tpu as pltpu

CH = 2000
NBUF = 16
LOOK = 12
IDX_BLK = 16384


def _body(f_ref, idx_ref, fb_ref, idxb_ref, out_ref, idxo_ref,
          bufs, rsems, wsems, *, nc, nf_full, c_mix, mix_f_rows, nidx):
    i = pl.program_id(0)
    n = nf_full * CH + mix_f_rows

    def start_read(c, static=None):
        b = jax.lax.rem(c, NBUF)
        pltpu.make_async_copy(
            fb_ref.at[pl.ds(c * CH - n, CH)], bufs.at[b], rsems.at[b]
        ).start()

    @pl.when(i == 0)
    def _():
        for c in range(LOOK):
            b = c % NBUF
            if c < nf_full:
                pltpu.make_async_copy(
                    f_ref.at[pl.ds(c * CH, CH)], bufs.at[b], rsems.at[b]
                ).start()
            elif c == c_mix:
                pltpu.make_async_copy(
                    f_ref.at[pl.ds(c * CH, mix_f_rows)],
                    bufs.at[b, pl.ds(0, mix_f_rows)], rsems.at[b]).start()
                pltpu.make_async_copy(
                    fb_ref.at[pl.ds(0, CH - mix_f_rows)],
                    bufs.at[b, pl.ds(mix_f_rows, CH - mix_f_rows)],
                    rsems.at[b]).start()
            else:
                start_read(c)

    j = i + LOOK
    bj = jax.lax.rem(j, NBUF)

    @pl.when(j < nc)
    def _():
        @pl.when(j >= NBUF)
        def _():
            pltpu.make_async_copy(
                fb_ref.at[pl.ds(0, CH)], out_ref.at[pl.ds(0, CH)],
                wsems.at[bj]).wait()
        start_read(j)

    bi = jax.lax.rem(i, NBUF)

    @pl.when(i != c_mix)
    def _():
        pltpu.make_async_copy(
            fb_ref.at[pl.ds(0, CH)], bufs.at[bi], rsems.at[bi]).wait()

    @pl.when(i == c_mix)
    def _():
        pltpu.make_async_copy(
            f_ref.at[pl.ds(0, mix_f_rows)],
            bufs.at[bi, pl.ds(0, mix_f_rows)], rsems.at[bi]).wait()
        pltpu.make_async_copy(
            fb_ref.at[pl.ds(0, CH - mix_f_rows)],
            bufs.at[bi, pl.ds(mix_f_rows, CH - mix_f_rows)],
            rsems.at[bi]).wait()

    pltpu.make_async_copy(
        bufs.at[bi], out_ref.at[pl.ds(i * CH, CH)], wsems.at[bi]).start()

    @pl.when(i == nc - 1)
    def _():
        for b in range(NBUF):
            pltpu.make_async_copy(
                fb_ref.at[pl.ds(0, CH)], out_ref.at[pl.ds(0, CH)],
                wsems.at[b]).wait()

    @pl.when(i == 0)
    def _():
        idxo_ref[...] = idx_ref[...]

    @pl.when((i > 0) & (i < nidx))
    def _():
        idxo_ref[...] = idxb_ref[...]


def kernel(f, idx, fb, idx_bank):
    f2 = f.reshape(-1, f.shape[-1])
    idx2 = idx.reshape(-1)
    N, F = f2.shape
    S = fb.shape[0]
    assert S % CH == 0
    nc = S // CH
    nf_full = N // CH
    mix_f_rows = N - nf_full * CH
    c_mix = nf_full if mix_f_rows else -1
    assert c_mix < LOOK
    nidx = pl.cdiv(S, IDX_BLK)
    nidx_in = nidx - 1

    body = functools.partial(
        _body, nc=nc, nf_full=nf_full, c_mix=c_mix,
        mix_f_rows=mix_f_rows, nidx=nidx,
    )

    out_fb, out_idx = pl.pallas_call(
        body,
        grid=(nc,),
        in_specs=[
            pl.BlockSpec(memory_space=pl.ANY),
            pl.BlockSpec((IDX_BLK,), lambda i: (0,)),
            pl.BlockSpec(memory_space=pl.ANY),
            pl.BlockSpec((IDX_BLK,), lambda i: (jnp.clip(i - 1, 0, nidx_in - 1),)),
        ],
        out_specs=[
            pl.BlockSpec(memory_space=pl.ANY),
            pl.BlockSpec((IDX_BLK,), lambda i: (jnp.minimum(i, nidx - 1),)),
        ],
        out_shape=[
            jax.ShapeDtypeStruct((S, F), fb.dtype),
            jax.ShapeDtypeStruct((S,), idx_bank.dtype),
        ],
        scratch_shapes=[
            pltpu.VMEM((NBUF, CH, F), fb.dtype),
            pltpu.SemaphoreType.DMA((NBUF,)),
            pltpu.SemaphoreType.DMA((NBUF,)),
        ],
    )(f2, idx2, fb, idx_bank)

    return (out_fb, out_idx)
